# Initial kernel scaffold; baseline (speedup 1.0000x reference)
import jax
import jax.numpy as jnp
from jax.experimental import pallas as pl


def kernel(x, edge_index, batch, params):
    raise NotImplementedError("write your pallas kernel here")


# speedup vs baseline: 6.1746x; 6.1746x over previous
import functools

import jax
import jax.numpy as jnp
from jax import lax
from jax.experimental import pallas as pl
from jax.experimental.pallas import tpu as pltpu, tpu_sc as plsc

N_NODES = 10000
N_EDGES = 320000
HID = 128
D_OUT = 64
N_GRAPHS = 64

NC = 2
NS = 16
NW = NC * NS
EPW = N_EDGES // NW
CH = 80
NCHUNK = EPW // CH
RPT = 640
N_PAD = NS * RPT


def _sc_aggr_body(h_hbm, src_hbm, dst_hbm, zeros_hbm, out_hbm,
                  sidx_v, didx_v, rows_v, aggr_sh, sem):
    cid = lax.axis_index("c")
    sid = lax.axis_index("s")
    wid = cid * NS + sid

    pltpu.sync_copy(zeros_hbm, aggr_sh.at[pl.ds(sid * RPT, RPT)])
    pltpu.sync_copy(src_hbm.at[wid], sidx_v)
    pltpu.sync_copy(dst_hbm.at[wid], didx_v)
    plsc.subcore_barrier()

    def chunk(j, carry):
        pltpu.async_copy(h_hbm.at[sidx_v.at[j]], rows_v, sem).wait()
        pltpu.sync_copy(rows_v, aggr_sh.at[didx_v.at[j]], add=True)
        return carry

    lax.fori_loop(0, NCHUNK, chunk, 0, unroll=False)
    plsc.subcore_barrier()
    pltpu.sync_copy(aggr_sh.at[pl.ds(sid * RPT, RPT)],
                    out_hbm.at[pl.ds(cid * N_PAD + sid * RPT, RPT)])


_sc_aggr_cache = []


def _sc_aggr(*args):
    if not _sc_aggr_cache:
        _sc_aggr_cache.append(functools.partial(
            pl.kernel,
            out_type=jax.ShapeDtypeStruct((NC * N_PAD, HID), jnp.float32),
            mesh=plsc.VectorSubcoreMesh(core_axis_name="c",
                                        subcore_axis_name="s"),
            scratch_types=[
                pltpu.VMEM((NCHUNK, CH), jnp.int32),
                pltpu.VMEM((NCHUNK, CH), jnp.int32),
                pltpu.VMEM((CH, HID), jnp.float32),
                pltpu.VMEM_SHARED((N_PAD, HID), jnp.float32),
                pltpu.SemaphoreType.DMA,
            ],
        )(_sc_aggr_body))
    return _sc_aggr_cache[0](*args)


_HI = lax.Precision.HIGHEST
_dot = functools.partial(jnp.dot, precision=_HI,
                         preferred_element_type=jnp.float32)
_segdot = functools.partial(
    lax.dot_general, dimension_numbers=(((0,), (0,)), ((), ())),
    precision=_HI, preferred_element_type=jnp.float32)


def _onehot(sb, eb):
    nid = lax.broadcasted_iota(jnp.int32, (N_NODES, N_GRAPHS), 0)
    return ((nid >= sb) & (nid < eb)).astype(jnp.float32)


def _mlp_body(h_ref, pp_ref, w1_ref, b1_ref, w2_ref, b2_ref, eps_ref,
              out_ref):
    out = (1.0 + eps_ref[0, 0]) * h_ref[...] + pp_ref[0] + pp_ref[1]
    out = jnp.maximum(_dot(out, w1_ref[...]) + b1_ref[...], 0.0)
    out_ref[...] = _dot(out, w2_ref[...]) + b2_ref[...]


_mlp = pl.pallas_call(
    _mlp_body,
    out_shape=jax.ShapeDtypeStruct((N_NODES, HID), jnp.float32),
)


def _gnorm_body(out_ref, sb_ref, eb_ref, sbt_ref, ebt_ref,
                gnw_ref, gnb_ref, gns_ref, hout_ref):
    out = out_ref[...]
    onehot = _onehot(sb_ref[...], eb_ref[...])
    cnts = (ebt_ref[...] - sbt_ref[...]).astype(jnp.float32)
    inv_cnt = 1.0 / jnp.maximum(cnts, 1.0)

    mean = _segdot(onehot, out) * inv_cnt
    sub = out - _dot(onehot, mean) * gns_ref[...]
    var = _segdot(onehot, sub * sub) * inv_cnt
    inv_std = 1.0 / jnp.sqrt(var + 1e-08)
    res = gnw_ref[...] * sub * _dot(onehot, inv_std) + gnb_ref[...]
    hout_ref[...] = jnp.maximum(res, 0.0)


_gnorm = pl.pallas_call(
    _gnorm_body,
    out_shape=jax.ShapeDtypeStruct((N_NODES, HID), jnp.float32),
)


def _proj_body(h_ref, sb_ref, eb_ref, pw1_ref, pb1_ref, pw2_ref, pb2_ref,
               zg_ref):
    z = jnp.maximum(_dot(h_ref[...], pw1_ref[...]) + pb1_ref[...], 0.0)
    z = _dot(z, pw2_ref[...]) + pb2_ref[...]
    zg_ref[...] = _segdot(_onehot(sb_ref[...], eb_ref[...]), z)


_proj = pl.pallas_call(
    _proj_body,
    out_shape=jax.ShapeDtypeStruct((N_GRAPHS, D_OUT), jnp.float32),
)


def kernel(x, edge_index, batch, params):
    src = edge_index[0].reshape(NW, NCHUNK, CH)
    dst = edge_index[1].reshape(NW, NCHUNK, CH)
    zeros_blk = jnp.zeros((RPT, HID), jnp.float32)
    bounds = jnp.searchsorted(batch, jnp.arange(N_GRAPHS + 1, dtype=jnp.int32),
                              side='left').astype(jnp.int32)
    sb = bounds[:N_GRAPHS].reshape(1, N_GRAPHS)
    eb = bounds[1:].reshape(1, N_GRAPHS)
    sbt = bounds[:N_GRAPHS].reshape(N_GRAPHS, 1)
    ebt = bounds[1:].reshape(N_GRAPHS, 1)

    h = x
    z_cat = []
    for l in range(3):
        p = params[f'l{l}']
        partials = _sc_aggr(h, src, dst, zeros_blk)
        partials = partials.reshape(NC, N_PAD, HID)[:, :N_NODES, :]
        out = _mlp(
            h, partials,
            p['w1'], p['b1'].reshape(1, HID),
            p['w2'], p['b2'].reshape(1, HID),
            p['eps'].reshape(1, 1),
        )
        h = _gnorm(
            out, sb, eb, sbt, ebt,
            p['gn_weight'].reshape(1, HID),
            p['gn_bias'].reshape(1, HID),
            p['gn_scale'].reshape(1, HID),
        )
        zg = _proj(h, sb, eb, p['pw1'], p['pb1'].reshape(1, HID),
                   p['pw2'], p['pb2'].reshape(1, D_OUT))
        z_cat.append(zg)
    return jnp.concatenate(z_cat, axis=-1)


# speedup vs baseline: 8.9375x; 1.4475x over previous
import functools

import jax
import jax.numpy as jnp
from jax import lax
from jax.experimental import pallas as pl
from jax.experimental.pallas import tpu as pltpu, tpu_sc as plsc

N_NODES = 10000
N_EDGES = 320000
HID = 128
D_OUT = 64
N_GRAPHS = 64

NC = 2
NS = 16
NW = NC * NS
EPW = N_EDGES // NW
CH = 80
NCHUNK = EPW // CH
RPT = 632
N_PAD = NS * RPT


NBUF = 2


def _sc_aggr_body(h_hbm, src_hbm, dst_hbm, zeros_hbm, out_hbm,
                  sidx_v, didx_v, rows_v, aggr_sh, sem0, sem1, isem0, isem1):
    sems = (sem0, sem1)
    isems = (isem0, isem1)
    cid = lax.axis_index("c")
    sid = lax.axis_index("s")
    wid = cid * NS + sid

    pltpu.sync_copy(zeros_hbm, aggr_sh.at[pl.ds(sid * RPT, RPT)])
    pltpu.sync_copy(src_hbm.at[wid], sidx_v)
    plsc.subcore_barrier()

    for b in range(NBUF):
        pltpu.async_copy(dst_hbm.at[wid, b], didx_v.at[b], isems[b])
        pltpu.async_copy(h_hbm.at[sidx_v.at[b]], rows_v.at[b], sems[b])

    def step(j, b):
        pltpu.make_async_copy(h_hbm.at[sidx_v.at[j]], rows_v.at[b],
                              sems[b]).wait()
        pltpu.make_async_copy(dst_hbm.at[wid, j], didx_v.at[b],
                              isems[b]).wait()
        pltpu.sync_copy(rows_v.at[b], aggr_sh.at[didx_v.at[b, 0]], add=True)
        nj = j + NBUF

        @pl.when(nj < NCHUNK)
        def _():
            pltpu.async_copy(dst_hbm.at[wid, nj], didx_v.at[b], isems[b])
            pltpu.async_copy(h_hbm.at[sidx_v.at[nj]], rows_v.at[b], sems[b])

    def outer(o, carry):
        for b in range(NBUF):
            step(o * NBUF + b, b)
        return carry

    lax.fori_loop(0, NCHUNK // NBUF, outer, 0, unroll=False)
    for j in range((NCHUNK // NBUF) * NBUF, NCHUNK):
        step(j, j % NBUF)
    plsc.subcore_barrier()
    pltpu.sync_copy(aggr_sh.at[pl.ds(sid * RPT, RPT)],
                    out_hbm.at[pl.ds(cid * N_PAD + sid * RPT, RPT)])


_sc_aggr_cache = []


def _sc_aggr(*args):
    if not _sc_aggr_cache:
        _sc_aggr_cache.append(functools.partial(
            pl.kernel,
            out_type=jax.ShapeDtypeStruct((NC * N_PAD, HID), jnp.float32),
            mesh=plsc.VectorSubcoreMesh(core_axis_name="c",
                                        subcore_axis_name="s"),
            scratch_types=[
                pltpu.VMEM((NCHUNK, CH), jnp.int32),
                pltpu.VMEM((NBUF, 1, CH), jnp.int32),
                pltpu.VMEM((NBUF, CH, HID), jnp.float32),
                pltpu.VMEM_SHARED((N_PAD, HID), jnp.float32),
            ] + [pltpu.SemaphoreType.DMA] * (2 * NBUF),
        )(_sc_aggr_body))
    return _sc_aggr_cache[0](*args)


_HI = lax.Precision.HIGHEST
_dot = functools.partial(jnp.dot, precision=_HI,
                         preferred_element_type=jnp.float32)
_segdot = functools.partial(
    lax.dot_general, dimension_numbers=(((0,), (0,)), ((), ())),
    precision=_HI, preferred_element_type=jnp.float32)


def _onehot(sb, eb):
    nid = lax.broadcasted_iota(jnp.int32, (N_NODES, N_GRAPHS), 0)
    return ((nid >= sb) & (nid < eb)).astype(jnp.float32)


def _mlp_body(h_ref, pp_ref, w1_ref, b1_ref, w2_ref, b2_ref, eps_ref,
              out_ref):
    out = (1.0 + eps_ref[0, 0]) * h_ref[...] + pp_ref[0] + pp_ref[1]
    out = jnp.maximum(_dot(out, w1_ref[...]) + b1_ref[...], 0.0)
    out_ref[...] = _dot(out, w2_ref[...]) + b2_ref[...]


_mlp = pl.pallas_call(
    _mlp_body,
    out_shape=jax.ShapeDtypeStruct((N_NODES, HID), jnp.float32),
)


def _gnorm_body(out_ref, sb_ref, eb_ref, sbt_ref, ebt_ref,
                gnw_ref, gnb_ref, gns_ref, hout_ref):
    out = out_ref[...]
    onehot = _onehot(sb_ref[...], eb_ref[...])
    cnts = (ebt_ref[...] - sbt_ref[...]).astype(jnp.float32)
    inv_cnt = 1.0 / jnp.maximum(cnts, 1.0)

    mean = _segdot(onehot, out) * inv_cnt
    sub = out - _dot(onehot, mean) * gns_ref[...]
    var = _segdot(onehot, sub * sub) * inv_cnt
    inv_std = 1.0 / jnp.sqrt(var + 1e-08)
    res = gnw_ref[...] * sub * _dot(onehot, inv_std) + gnb_ref[...]
    hout_ref[...] = jnp.maximum(res, 0.0)


_gnorm = pl.pallas_call(
    _gnorm_body,
    out_shape=jax.ShapeDtypeStruct((N_NODES, HID), jnp.float32),
)


def _proj_body(h_ref, sb_ref, eb_ref, pw1_ref, pb1_ref, pw2_ref, pb2_ref,
               zg_ref):
    z = jnp.maximum(_dot(h_ref[...], pw1_ref[...]) + pb1_ref[...], 0.0)
    z = _dot(z, pw2_ref[...]) + pb2_ref[...]
    zg_ref[...] = _segdot(_onehot(sb_ref[...], eb_ref[...]), z)


_proj = pl.pallas_call(
    _proj_body,
    out_shape=jax.ShapeDtypeStruct((N_GRAPHS, D_OUT), jnp.float32),
)


def kernel(x, edge_index, batch, params):
    src = edge_index[0].reshape(NW, NCHUNK, CH)
    dst = edge_index[1].reshape(NW, NCHUNK, 1, CH)
    zeros_blk = jnp.zeros((RPT, HID), jnp.float32)
    bounds = jnp.searchsorted(batch, jnp.arange(N_GRAPHS + 1, dtype=jnp.int32),
                              side='left').astype(jnp.int32)
    sb = bounds[:N_GRAPHS].reshape(1, N_GRAPHS)
    eb = bounds[1:].reshape(1, N_GRAPHS)
    sbt = bounds[:N_GRAPHS].reshape(N_GRAPHS, 1)
    ebt = bounds[1:].reshape(N_GRAPHS, 1)

    h = x
    z_cat = []
    for l in range(3):
        p = params[f'l{l}']
        partials = _sc_aggr(h, src, dst, zeros_blk)
        partials = partials.reshape(NC, N_PAD, HID)[:, :N_NODES, :]
        out = _mlp(
            h, partials,
            p['w1'], p['b1'].reshape(1, HID),
            p['w2'], p['b2'].reshape(1, HID),
            p['eps'].reshape(1, 1),
        )
        h = _gnorm(
            out, sb, eb, sbt, ebt,
            p['gn_weight'].reshape(1, HID),
            p['gn_bias'].reshape(1, HID),
            p['gn_scale'].reshape(1, HID),
        )
        zg = _proj(h, sb, eb, p['pw1'], p['pb1'].reshape(1, HID),
                   p['pw2'], p['pb2'].reshape(1, D_OUT))
        z_cat.append(zg)
    return jnp.concatenate(z_cat, axis=-1)


# speedup vs baseline: 11.1711x; 1.2499x over previous
import functools

import jax
import jax.numpy as jnp
from jax import lax
from jax.experimental import pallas as pl
from jax.experimental.pallas import tpu as pltpu, tpu_sc as plsc

N_NODES = 10000
N_EDGES = 320000
HID = 128
D_OUT = 64
N_GRAPHS = 64

NC = 2
NS = 16
NW = NC * NS
EPW = N_EDGES // NW
CH = 80
NCHUNK = EPW // CH
RPT = 632
N_PAD = NS * RPT


NBUF = 2


def _sc_aggr_body(h_hbm, src_hbm, dst_hbm, zeros_hbm, out_hbm,
                  sidx_v, didx_v, rows_v, aggr_sh, sem0, sem1, isem0, isem1):
    sems = (sem0, sem1)
    isems = (isem0, isem1)
    cid = lax.axis_index("c")
    sid = lax.axis_index("s")
    wid = cid * NS + sid

    pltpu.sync_copy(zeros_hbm, aggr_sh.at[pl.ds(sid * RPT, RPT)])
    pltpu.sync_copy(src_hbm.at[wid], sidx_v)
    plsc.subcore_barrier()

    for b in range(NBUF):
        pltpu.async_copy(dst_hbm.at[wid, b], didx_v.at[b], isems[b])
        pltpu.async_copy(h_hbm.at[sidx_v.at[b]], rows_v.at[b], sems[b])

    def step(j, b):
        pltpu.make_async_copy(h_hbm.at[sidx_v.at[j]], rows_v.at[b],
                              sems[b]).wait()
        pltpu.make_async_copy(dst_hbm.at[wid, j], didx_v.at[b],
                              isems[b]).wait()
        pltpu.sync_copy(rows_v.at[b], aggr_sh.at[didx_v.at[b, 0]], add=True)
        nj = j + NBUF

        @pl.when(nj < NCHUNK)
        def _():
            pltpu.async_copy(dst_hbm.at[wid, nj], didx_v.at[b], isems[b])
            pltpu.async_copy(h_hbm.at[sidx_v.at[nj]], rows_v.at[b], sems[b])

    def outer(o, carry):
        for b in range(NBUF):
            step(o * NBUF + b, b)
        return carry

    lax.fori_loop(0, NCHUNK // NBUF, outer, 0, unroll=False)
    for j in range((NCHUNK // NBUF) * NBUF, NCHUNK):
        step(j, j % NBUF)
    plsc.subcore_barrier()
    pltpu.sync_copy(aggr_sh.at[pl.ds(sid * RPT, RPT)],
                    out_hbm.at[pl.ds(cid * N_PAD + sid * RPT, RPT)])


_sc_aggr_cache = []


def _sc_aggr(*args):
    if not _sc_aggr_cache:
        _sc_aggr_cache.append(functools.partial(
            pl.kernel,
            out_type=jax.ShapeDtypeStruct((NC * N_PAD, HID), jnp.float32),
            mesh=plsc.VectorSubcoreMesh(core_axis_name="c",
                                        subcore_axis_name="s"),
            scratch_types=[
                pltpu.VMEM((NCHUNK, CH), jnp.int32),
                pltpu.VMEM((NBUF, 1, CH), jnp.int32),
                pltpu.VMEM((NBUF, CH, HID), jnp.float32),
                pltpu.VMEM_SHARED((N_PAD, HID), jnp.float32),
            ] + [pltpu.SemaphoreType.DMA] * (2 * NBUF),
        )(_sc_aggr_body))
    return _sc_aggr_cache[0](*args)


_dot = functools.partial(jnp.dot, preferred_element_type=jnp.float32)
_segdot = functools.partial(
    lax.dot_general, dimension_numbers=(((0,), (0,)), ((), ())),
    preferred_element_type=jnp.float32)


def _onehot(sb, eb):
    nid = lax.broadcasted_iota(jnp.int32, (N_NODES, N_GRAPHS), 0)
    return ((nid >= sb) & (nid < eb)).astype(jnp.float32)


def _mlp_body(h_ref, pp_ref, w1_ref, b1_ref, w2_ref, b2_ref, eps_ref,
              out_ref):
    out = (1.0 + eps_ref[0, 0]) * h_ref[...] + pp_ref[0] + pp_ref[1]
    out = jnp.maximum(_dot(out, w1_ref[...]) + b1_ref[...], 0.0)
    out_ref[...] = _dot(out, w2_ref[...]) + b2_ref[...]


_mlp = pl.pallas_call(
    _mlp_body,
    out_shape=jax.ShapeDtypeStruct((N_NODES, HID), jnp.float32),
)


def _gnorm_body(out_ref, sb_ref, eb_ref, sbt_ref, ebt_ref,
                gnw_ref, gnb_ref, gns_ref,
                pw1_ref, pb1_ref, pw2_ref, pb2_ref, hout_ref, zg_ref):
    out = out_ref[...]
    onehot = _onehot(sb_ref[...], eb_ref[...])
    cnts = (ebt_ref[...] - sbt_ref[...]).astype(jnp.float32)
    inv_cnt = 1.0 / jnp.maximum(cnts, 1.0)

    mean = _segdot(onehot, out) * inv_cnt
    sub = out - _dot(onehot, mean) * gns_ref[...]
    var = _segdot(onehot, sub * sub) * inv_cnt
    inv_std = 1.0 / jnp.sqrt(var + 1e-08)
    res = gnw_ref[...] * sub * _dot(onehot, inv_std) + gnb_ref[...]
    hn = jnp.maximum(res, 0.0)
    hout_ref[...] = hn
    z = jnp.maximum(_dot(hn, pw1_ref[...]) + pb1_ref[...], 0.0)
    z = _dot(z, pw2_ref[...]) + pb2_ref[...]
    zg_ref[...] = _segdot(onehot, z)


_gnorm = pl.pallas_call(
    _gnorm_body,
    out_shape=(
        jax.ShapeDtypeStruct((N_NODES, HID), jnp.float32),
        jax.ShapeDtypeStruct((N_GRAPHS, D_OUT), jnp.float32),
    ),
)


def kernel(x, edge_index, batch, params):
    src = edge_index[0].reshape(NW, NCHUNK, CH)
    dst = edge_index[1].reshape(NW, NCHUNK, 1, CH)
    zeros_blk = jnp.zeros((RPT, HID), jnp.float32)
    bounds = jnp.searchsorted(batch, jnp.arange(N_GRAPHS + 1, dtype=jnp.int32),
                              side='left').astype(jnp.int32)
    sb = bounds[:N_GRAPHS].reshape(1, N_GRAPHS)
    eb = bounds[1:].reshape(1, N_GRAPHS)
    sbt = bounds[:N_GRAPHS].reshape(N_GRAPHS, 1)
    ebt = bounds[1:].reshape(N_GRAPHS, 1)

    h = x
    z_cat = []
    for l in range(3):
        p = params[f'l{l}']
        partials = _sc_aggr(h, src, dst, zeros_blk)
        partials = partials.reshape(NC, N_PAD, HID)[:, :N_NODES, :]
        out = _mlp(
            h, partials,
            p['w1'], p['b1'].reshape(1, HID),
            p['w2'], p['b2'].reshape(1, HID),
            p['eps'].reshape(1, 1),
        )
        h, zg = _gnorm(
            out, sb, eb, sbt, ebt,
            p['gn_weight'].reshape(1, HID),
            p['gn_bias'].reshape(1, HID),
            p['gn_scale'].reshape(1, HID),
            p['pw1'], p['pb1'].reshape(1, HID),
            p['pw2'], p['pb2'].reshape(1, D_OUT),
        )
        z_cat.append(zg)
    return jnp.concatenate(z_cat, axis=-1)


# speedup vs baseline: 11.2036x; 1.0029x over previous
import functools

import jax
import jax.numpy as jnp
from jax import lax
from jax.experimental import pallas as pl
from jax.experimental.pallas import tpu as pltpu, tpu_sc as plsc

N_NODES = 10000
N_EDGES = 320000
HID = 128
D_OUT = 64
N_GRAPHS = 64

NC = 2
NS = 16
NW = NC * NS
EPW = N_EDGES // NW
CH = 80
NCHUNK = EPW // CH
RPT = 632
N_PAD = NS * RPT


NBUF = 3


def _sc_aggr_body(h_hbm, src_hbm, dst_hbm, zeros_hbm, out_hbm,
                  sidx_v, didx_v, rows_v, aggr_sh,
                  sem0, sem1, sem2, isem0, isem1, isem2,
                  jsem0, jsem1, jsem2):
    sems = (sem0, sem1, sem2)
    isems = (isem0, isem1, isem2)
    jsems = (jsem0, jsem1, jsem2)
    cid = lax.axis_index("c")
    sid = lax.axis_index("s")
    wid = cid * NS + sid

    pltpu.sync_copy(zeros_hbm, aggr_sh.at[pl.ds(sid * RPT, RPT)])
    plsc.subcore_barrier()

    for b in range(NBUF):
        pltpu.async_copy(dst_hbm.at[wid, b], didx_v.at[b], isems[b])
        pltpu.async_copy(src_hbm.at[wid, b], sidx_v.at[b], jsems[b])
    for b in range(NBUF - 1):
        pltpu.make_async_copy(src_hbm.at[wid, b], sidx_v.at[b],
                              jsems[b]).wait()
        pltpu.async_copy(h_hbm.at[sidx_v.at[b, 0]], rows_v.at[b], sems[b])

    def step(j, b):
        pltpu.make_async_copy(dst_hbm.at[wid, j], didx_v.at[b],
                              isems[b]).wait()
        pltpu.make_async_copy(h_hbm.at[sidx_v.at[b, 0]], rows_v.at[b],
                              sems[b]).wait()
        pltpu.sync_copy(rows_v.at[b], aggr_sh.at[didx_v.at[b, 0]], add=True)
        nj = j + NBUF

        @pl.when(nj < NCHUNK)
        def _():
            pltpu.async_copy(dst_hbm.at[wid, nj], didx_v.at[b], isems[b])
            pltpu.async_copy(src_hbm.at[wid, nj], sidx_v.at[b], jsems[b])
        ng = j + NBUF - 1
        b2 = (b + NBUF - 1) % NBUF

        @pl.when(ng < NCHUNK)
        def _():
            pltpu.make_async_copy(src_hbm.at[wid, ng], sidx_v.at[b2],
                                  jsems[b2]).wait()
            pltpu.async_copy(h_hbm.at[sidx_v.at[b2, 0]], rows_v.at[b2],
                             sems[b2])

    def outer(o, carry):
        for b in range(NBUF):
            step(o * NBUF + b, b)
        return carry

    lax.fori_loop(0, NCHUNK // NBUF, outer, 0, unroll=False)
    for j in range((NCHUNK // NBUF) * NBUF, NCHUNK):
        step(j, j % NBUF)
    plsc.subcore_barrier()
    pltpu.sync_copy(aggr_sh.at[pl.ds(sid * RPT, RPT)],
                    out_hbm.at[pl.ds(cid * N_PAD + sid * RPT, RPT)])


_sc_aggr_cache = []


def _sc_aggr(*args):
    if not _sc_aggr_cache:
        _sc_aggr_cache.append(functools.partial(
            pl.kernel,
            out_type=jax.ShapeDtypeStruct((NC * N_PAD, HID), jnp.float32),
            mesh=plsc.VectorSubcoreMesh(core_axis_name="c",
                                        subcore_axis_name="s"),
            scratch_types=[
                pltpu.VMEM((NBUF, 1, CH), jnp.int32),
                pltpu.VMEM((NBUF, 1, CH), jnp.int32),
                pltpu.VMEM((NBUF, CH, HID), jnp.float32),
                pltpu.VMEM_SHARED((N_PAD, HID), jnp.float32),
            ] + [pltpu.SemaphoreType.DMA] * (3 * NBUF),
        )(_sc_aggr_body))
    return _sc_aggr_cache[0](*args)


_dot = functools.partial(jnp.dot, preferred_element_type=jnp.float32)
_segdot = functools.partial(
    lax.dot_general, dimension_numbers=(((0,), (0,)), ((), ())),
    preferred_element_type=jnp.float32)


def _onehot(sb, eb):
    nid = lax.broadcasted_iota(jnp.int32, (N_NODES, N_GRAPHS), 0)
    return ((nid >= sb) & (nid < eb)).astype(jnp.float32)


def _mlp_body(h_ref, pp_ref, w1_ref, b1_ref, w2_ref, b2_ref, eps_ref,
              out_ref):
    out = (1.0 + eps_ref[0, 0]) * h_ref[...] + pp_ref[0] + pp_ref[1]
    out = jnp.maximum(_dot(out, w1_ref[...]) + b1_ref[...], 0.0)
    out_ref[...] = _dot(out, w2_ref[...]) + b2_ref[...]


_mlp = pl.pallas_call(
    _mlp_body,
    out_shape=jax.ShapeDtypeStruct((N_NODES, HID), jnp.float32),
)


def _gnorm_body(out_ref, sb_ref, eb_ref, sbt_ref, ebt_ref,
                gnw_ref, gnb_ref, gns_ref,
                pw1_ref, pb1_ref, pw2_ref, pb2_ref, hout_ref, zg_ref):
    out = out_ref[...]
    onehot = _onehot(sb_ref[...], eb_ref[...])
    cnts = (ebt_ref[...] - sbt_ref[...]).astype(jnp.float32)
    inv_cnt = 1.0 / jnp.maximum(cnts, 1.0)

    mean = _segdot(onehot, out) * inv_cnt
    sub = out - _dot(onehot, mean) * gns_ref[...]
    var = _segdot(onehot, sub * sub) * inv_cnt
    inv_std = 1.0 / jnp.sqrt(var + 1e-08)
    res = gnw_ref[...] * sub * _dot(onehot, inv_std) + gnb_ref[...]
    hn = jnp.maximum(res, 0.0)
    hout_ref[...] = hn
    z = jnp.maximum(_dot(hn, pw1_ref[...]) + pb1_ref[...], 0.0)
    z = _dot(z, pw2_ref[...]) + pb2_ref[...]
    zg_ref[...] = _segdot(onehot, z)


_gnorm = pl.pallas_call(
    _gnorm_body,
    out_shape=(
        jax.ShapeDtypeStruct((N_NODES, HID), jnp.float32),
        jax.ShapeDtypeStruct((N_GRAPHS, D_OUT), jnp.float32),
    ),
)


def kernel(x, edge_index, batch, params):
    src = edge_index[0].reshape(NW, NCHUNK, 1, CH)
    dst = edge_index[1].reshape(NW, NCHUNK, 1, CH)
    zeros_blk = jnp.zeros((RPT, HID), jnp.float32)
    bounds = jnp.searchsorted(batch, jnp.arange(N_GRAPHS + 1, dtype=jnp.int32),
                              side='left').astype(jnp.int32)
    sb = bounds[:N_GRAPHS].reshape(1, N_GRAPHS)
    eb = bounds[1:].reshape(1, N_GRAPHS)
    sbt = bounds[:N_GRAPHS].reshape(N_GRAPHS, 1)
    ebt = bounds[1:].reshape(N_GRAPHS, 1)

    h = x
    z_cat = []
    for l in range(3):
        p = params[f'l{l}']
        partials = _sc_aggr(h, src, dst, zeros_blk)
        partials = partials.reshape(NC, N_PAD, HID)[:, :N_NODES, :]
        out = _mlp(
            h, partials,
            p['w1'], p['b1'].reshape(1, HID),
            p['w2'], p['b2'].reshape(1, HID),
            p['eps'].reshape(1, 1),
        )
        h, zg = _gnorm(
            out, sb, eb, sbt, ebt,
            p['gn_weight'].reshape(1, HID),
            p['gn_bias'].reshape(1, HID),
            p['gn_scale'].reshape(1, HID),
            p['pw1'], p['pb1'].reshape(1, HID),
            p['pw2'], p['pb2'].reshape(1, D_OUT),
        )
        z_cat.append(zg)
    return jnp.concatenate(z_cat, axis=-1)


# speedup vs baseline: 11.5902x; 1.0345x over previous
import functools

import jax
import jax.numpy as jnp
from jax import lax
from jax.experimental import pallas as pl
from jax.experimental.pallas import tpu as pltpu, tpu_sc as plsc

N_NODES = 10000
N_EDGES = 320000
HID = 128
D_OUT = 64
N_GRAPHS = 64

NC = 2
NS = 16
NW = NC * NS
EPW = N_EDGES // NW
CH = 80
NCHUNK = EPW // CH
RPT = 632
N_PAD = NS * RPT


NBUF = 3


def _sc_aggr_body(h_hbm, src_hbm, dst_hbm, zeros_hbm, out_hbm,
                  sidx_v, didx_v, rows_v, aggr_sh,
                  sem0, sem1, sem2, isem0, isem1, isem2,
                  jsem0, jsem1, jsem2):
    sems = (sem0, sem1, sem2)
    isems = (isem0, isem1, isem2)
    jsems = (jsem0, jsem1, jsem2)
    cid = lax.axis_index("c")
    sid = lax.axis_index("s")
    wid = cid * NS + sid

    pltpu.sync_copy(zeros_hbm, aggr_sh.at[pl.ds(sid * RPT, RPT)])
    plsc.subcore_barrier()

    for b in range(NBUF):
        pltpu.async_copy(dst_hbm.at[wid, b], didx_v.at[b], isems[b])
        pltpu.async_copy(src_hbm.at[wid, b], sidx_v.at[b], jsems[b])
    for b in range(NBUF - 1):
        pltpu.make_async_copy(src_hbm.at[wid, b], sidx_v.at[b],
                              jsems[b]).wait()
        pltpu.async_copy(h_hbm.at[sidx_v.at[b, 0]], rows_v.at[b], sems[b])

    def step(j, b):
        pltpu.make_async_copy(dst_hbm.at[wid, j], didx_v.at[b],
                              isems[b]).wait()
        pltpu.make_async_copy(h_hbm.at[sidx_v.at[b, 0]], rows_v.at[b],
                              sems[b]).wait()
        pltpu.sync_copy(rows_v.at[b], aggr_sh.at[didx_v.at[b, 0]], add=True)
        nj = j + NBUF

        @pl.when(nj < NCHUNK)
        def _():
            pltpu.async_copy(dst_hbm.at[wid, nj], didx_v.at[b], isems[b])
            pltpu.async_copy(src_hbm.at[wid, nj], sidx_v.at[b], jsems[b])
        ng = j + NBUF - 1
        b2 = (b + NBUF - 1) % NBUF

        @pl.when(ng < NCHUNK)
        def _():
            pltpu.make_async_copy(src_hbm.at[wid, ng], sidx_v.at[b2],
                                  jsems[b2]).wait()
            pltpu.async_copy(h_hbm.at[sidx_v.at[b2, 0]], rows_v.at[b2],
                             sems[b2])

    def outer(o, carry):
        for b in range(NBUF):
            step(o * NBUF + b, b)
        return carry

    lax.fori_loop(0, NCHUNK // NBUF, outer, 0, unroll=False)
    for j in range((NCHUNK // NBUF) * NBUF, NCHUNK):
        step(j, j % NBUF)
    plsc.subcore_barrier()
    pltpu.sync_copy(aggr_sh.at[pl.ds(sid * RPT, RPT)],
                    out_hbm.at[pl.ds(cid * N_PAD + sid * RPT, RPT)])


_sc_aggr_cache = []


def _sc_aggr(*args):
    if not _sc_aggr_cache:
        _sc_aggr_cache.append(functools.partial(
            pl.kernel,
            out_type=jax.ShapeDtypeStruct((NC * N_PAD, HID), jnp.float32),
            mesh=plsc.VectorSubcoreMesh(core_axis_name="c",
                                        subcore_axis_name="s"),
            scratch_types=[
                pltpu.VMEM((NBUF, 1, CH), jnp.int32),
                pltpu.VMEM((NBUF, 1, CH), jnp.int32),
                pltpu.VMEM((NBUF, CH, HID), jnp.float32),
                pltpu.VMEM_SHARED((N_PAD, HID), jnp.float32),
            ] + [pltpu.SemaphoreType.DMA] * (3 * NBUF),
        )(_sc_aggr_body))
    return _sc_aggr_cache[0](*args)


_dot = functools.partial(jnp.dot, preferred_element_type=jnp.float32)
_segdot = functools.partial(
    lax.dot_general, dimension_numbers=(((0,), (0,)), ((), ())),
    preferred_element_type=jnp.float32)


def _onehot(sb, eb):
    nid = lax.broadcasted_iota(jnp.int32, (N_NODES, N_GRAPHS), 0)
    return ((nid >= sb) & (nid < eb)).astype(jnp.float32)


def _gnorm_body(h_ref, pp_ref, w1_ref, b1_ref, w2_ref, b2_ref, eps_ref,
                sb_ref, eb_ref, sbt_ref, ebt_ref,
                gnw_ref, gnb_ref, gns_ref,
                pw1_ref, pb1_ref, pw2_ref, pb2_ref, hout_ref, zg_ref):
    out = (1.0 + eps_ref[0, 0]) * h_ref[...] + pp_ref[0] + pp_ref[1]
    out = jnp.maximum(_dot(out, w1_ref[...]) + b1_ref[...], 0.0)
    out = _dot(out, w2_ref[...]) + b2_ref[...]
    onehot = _onehot(sb_ref[...], eb_ref[...])
    cnts = (ebt_ref[...] - sbt_ref[...]).astype(jnp.float32)
    inv_cnt = 1.0 / jnp.maximum(cnts, 1.0)

    mean = _segdot(onehot, out) * inv_cnt
    sub = out - _dot(onehot, mean) * gns_ref[...]
    var = _segdot(onehot, sub * sub) * inv_cnt
    inv_std = 1.0 / jnp.sqrt(var + 1e-08)
    res = gnw_ref[...] * sub * _dot(onehot, inv_std) + gnb_ref[...]
    hn = jnp.maximum(res, 0.0)
    hout_ref[...] = hn
    z = jnp.maximum(_dot(hn, pw1_ref[...]) + pb1_ref[...], 0.0)
    z = _dot(z, pw2_ref[...]) + pb2_ref[...]
    zg_ref[...] = _segdot(onehot, z)


_gnorm = pl.pallas_call(
    _gnorm_body,
    out_shape=(
        jax.ShapeDtypeStruct((N_NODES, HID), jnp.float32),
        jax.ShapeDtypeStruct((N_GRAPHS, D_OUT), jnp.float32),
    ),
)


def kernel(x, edge_index, batch, params):
    src = edge_index[0].reshape(NW, NCHUNK, 1, CH)
    dst = edge_index[1].reshape(NW, NCHUNK, 1, CH)
    zeros_blk = jnp.zeros((RPT, HID), jnp.float32)
    bounds = jnp.searchsorted(batch, jnp.arange(N_GRAPHS + 1, dtype=jnp.int32),
                              side='left').astype(jnp.int32)
    sb = bounds[:N_GRAPHS].reshape(1, N_GRAPHS)
    eb = bounds[1:].reshape(1, N_GRAPHS)
    sbt = bounds[:N_GRAPHS].reshape(N_GRAPHS, 1)
    ebt = bounds[1:].reshape(N_GRAPHS, 1)

    h = x
    z_cat = []
    for l in range(3):
        p = params[f'l{l}']
        partials = _sc_aggr(h, src, dst, zeros_blk)
        partials = partials.reshape(NC, N_PAD, HID)[:, :N_NODES, :]
        h, zg = _gnorm(
            h, partials,
            p['w1'], p['b1'].reshape(1, HID),
            p['w2'], p['b2'].reshape(1, HID),
            p['eps'].reshape(1, 1),
            sb, eb, sbt, ebt,
            p['gn_weight'].reshape(1, HID),
            p['gn_bias'].reshape(1, HID),
            p['gn_scale'].reshape(1, HID),
            p['pw1'], p['pb1'].reshape(1, HID),
            p['pw2'], p['pb2'].reshape(1, D_OUT),
        )
        z_cat.append(zg)
    return jnp.concatenate(z_cat, axis=-1)


# speedup vs baseline: 11.9757x; 1.0333x over previous
import functools

import jax
import jax.numpy as jnp
from jax import lax
from jax.experimental import pallas as pl
from jax.experimental.pallas import tpu as pltpu, tpu_sc as plsc

N_NODES = 10000
N_EDGES = 320000
HID = 128
D_OUT = 64
N_GRAPHS = 64

NC = 2
NS = 16
NW = NC * NS
EPW = N_EDGES // NW
CH = 80
NCHUNK = EPW // CH
RPT = 632
N_PAD = NS * RPT


NBUF = 3


def _sc_aggr_body(h_hbm, src_hbm, dst_hbm, zeros_hbm, out_hbm,
                  sidx_v, didx_v, rows_v, aggr_sh,
                  sem0, sem1, sem2, isem0, isem1, isem2,
                  jsem0, jsem1, jsem2):
    sems = (sem0, sem1, sem2)
    isems = (isem0, isem1, isem2)
    jsems = (jsem0, jsem1, jsem2)
    cid = lax.axis_index("c")
    sid = lax.axis_index("s")
    wid = cid * NS + sid

    pltpu.sync_copy(zeros_hbm, aggr_sh.at[pl.ds(sid * RPT, RPT)])
    plsc.subcore_barrier()

    for b in range(NBUF):
        pltpu.async_copy(dst_hbm.at[wid, b], didx_v.at[b], isems[b])
        pltpu.async_copy(src_hbm.at[wid, b], sidx_v.at[b], jsems[b])
    for b in range(NBUF - 1):
        pltpu.make_async_copy(src_hbm.at[wid, b], sidx_v.at[b],
                              jsems[b]).wait()
        pltpu.async_copy(h_hbm.at[sidx_v.at[b, 0]], rows_v.at[b], sems[b])

    def step(j, b):
        pltpu.make_async_copy(dst_hbm.at[wid, j], didx_v.at[b],
                              isems[b]).wait()
        pltpu.make_async_copy(h_hbm.at[sidx_v.at[b, 0]], rows_v.at[b],
                              sems[b]).wait()
        pltpu.sync_copy(rows_v.at[b], aggr_sh.at[didx_v.at[b, 0]], add=True)
        nj = j + NBUF

        @pl.when(nj < NCHUNK)
        def _():
            pltpu.async_copy(dst_hbm.at[wid, nj], didx_v.at[b], isems[b])
            pltpu.async_copy(src_hbm.at[wid, nj], sidx_v.at[b], jsems[b])
        ng = j + NBUF - 1
        b2 = (b + NBUF - 1) % NBUF

        @pl.when(ng < NCHUNK)
        def _():
            pltpu.make_async_copy(src_hbm.at[wid, ng], sidx_v.at[b2],
                                  jsems[b2]).wait()
            pltpu.async_copy(h_hbm.at[sidx_v.at[b2, 0]], rows_v.at[b2],
                             sems[b2])

    def outer(o, carry):
        for b in range(NBUF):
            step(o * NBUF + b, b)
        return carry

    lax.fori_loop(0, NCHUNK // NBUF, outer, 0, unroll=False)
    for j in range((NCHUNK // NBUF) * NBUF, NCHUNK):
        step(j, j % NBUF)
    plsc.subcore_barrier()
    pltpu.sync_copy(aggr_sh.at[pl.ds(sid * RPT, RPT)],
                    out_hbm.at[pl.ds(cid * N_PAD + sid * RPT, RPT)])


_sc_aggr_cache = []


def _sc_aggr(*args):
    if not _sc_aggr_cache:
        _sc_aggr_cache.append(functools.partial(
            pl.kernel,
            out_type=jax.ShapeDtypeStruct((NC * N_PAD, HID), jnp.float32),
            mesh=plsc.VectorSubcoreMesh(core_axis_name="c",
                                        subcore_axis_name="s"),
            scratch_types=[
                pltpu.VMEM((NBUF, 1, CH), jnp.int32),
                pltpu.VMEM((NBUF, 1, CH), jnp.int32),
                pltpu.VMEM((NBUF, CH, HID), jnp.float32),
                pltpu.VMEM_SHARED((N_PAD, HID), jnp.float32),
            ] + [pltpu.SemaphoreType.DMA] * (3 * NBUF),
        )(_sc_aggr_body))
    return _sc_aggr_cache[0](*args)


_dot = functools.partial(jnp.dot, preferred_element_type=jnp.float32)
_segdot = functools.partial(
    lax.dot_general, dimension_numbers=(((0,), (0,)), ((), ())),
    preferred_element_type=jnp.float32)


def _prep_body(sb_ref, eb_ref, sbt_ref, ebt_ref, oh_ref, oht_ref):
    nid = lax.broadcasted_iota(jnp.int32, (N_NODES, N_GRAPHS), 0)
    oh_ref[...] = ((nid >= sb_ref[...]) &
                   (nid < eb_ref[...])).astype(jnp.float32)
    nidt = lax.broadcasted_iota(jnp.int32, (N_GRAPHS, N_NODES), 1)
    oht_ref[...] = ((nidt >= sbt_ref[...]) &
                    (nidt < ebt_ref[...])).astype(jnp.float32)


_prep = pl.pallas_call(
    _prep_body,
    out_shape=(
        jax.ShapeDtypeStruct((N_NODES, N_GRAPHS), jnp.float32),
        jax.ShapeDtypeStruct((N_GRAPHS, N_NODES), jnp.float32),
    ),
)


def _gnorm_body(h_ref, pp_ref, w1_ref, b1_ref, w2_ref, b2_ref, eps_ref,
                oh_ref, oht_ref, sbt_ref, ebt_ref,
                gnw_ref, gnb_ref, gns_ref,
                pw1_ref, pb1_ref, pw2_ref, pb2_ref, hout_ref, zg_ref):
    out = (1.0 + eps_ref[0, 0]) * h_ref[...] \
        + pp_ref[0, :N_NODES] + pp_ref[1, :N_NODES]
    out = jnp.maximum(_dot(out, w1_ref[...]) + b1_ref[...], 0.0)
    out = _dot(out, w2_ref[...]) + b2_ref[...]
    onehot = oh_ref[...]
    oht = oht_ref[...]
    cnts = (ebt_ref[...] - sbt_ref[...]).astype(jnp.float32)
    inv_cnt = 1.0 / jnp.maximum(cnts, 1.0)

    mean = _dot(oht, out) * inv_cnt
    sub = out - _dot(onehot, mean) * gns_ref[...]
    var = _dot(oht, sub * sub) * inv_cnt
    inv_std = 1.0 / jnp.sqrt(var + 1e-08)
    res = gnw_ref[...] * sub * _dot(onehot, inv_std) + gnb_ref[...]
    hn = jnp.maximum(res, 0.0)
    hout_ref[...] = hn
    z = jnp.maximum(_dot(hn, pw1_ref[...]) + pb1_ref[...], 0.0)
    z = _dot(z, pw2_ref[...]) + pb2_ref[...]
    zg_ref[...] = _dot(oht, z)


_gnorm = pl.pallas_call(
    _gnorm_body,
    out_shape=(
        jax.ShapeDtypeStruct((N_NODES, HID), jnp.float32),
        jax.ShapeDtypeStruct((N_GRAPHS, D_OUT), jnp.float32),
    ),
)


def kernel(x, edge_index, batch, params):
    src = edge_index[0].reshape(NW, NCHUNK, 1, CH)
    dst = edge_index[1].reshape(NW, NCHUNK, 1, CH)
    zeros_blk = jnp.zeros((RPT, HID), jnp.float32)
    bounds = jnp.searchsorted(batch, jnp.arange(N_GRAPHS + 1, dtype=jnp.int32),
                              side='left').astype(jnp.int32)
    sb = bounds[:N_GRAPHS].reshape(1, N_GRAPHS)
    eb = bounds[1:].reshape(1, N_GRAPHS)
    sbt = bounds[:N_GRAPHS].reshape(N_GRAPHS, 1)
    ebt = bounds[1:].reshape(N_GRAPHS, 1)

    oh, oht = _prep(sb, eb, sbt, ebt)

    h = x
    z_cat = []
    for l in range(3):
        p = params[f'l{l}']
        partials = _sc_aggr(h, src, dst, zeros_blk).reshape(NC, N_PAD, HID)
        h, zg = _gnorm(
            h, partials,
            p['w1'], p['b1'].reshape(1, HID),
            p['w2'], p['b2'].reshape(1, HID),
            p['eps'].reshape(1, 1),
            oh, oht, sbt, ebt,
            p['gn_weight'].reshape(1, HID),
            p['gn_bias'].reshape(1, HID),
            p['gn_scale'].reshape(1, HID),
            p['pw1'], p['pb1'].reshape(1, HID),
            p['pw2'], p['pb2'].reshape(1, D_OUT),
        )
        z_cat.append(zg)
    return jnp.concatenate(z_cat, axis=-1)


# speedup vs baseline: 11.9856x; 1.0008x over previous
import functools

import jax
import jax.numpy as jnp
from jax import lax
from jax.experimental import pallas as pl
from jax.experimental.pallas import tpu as pltpu, tpu_sc as plsc

N_NODES = 10000
N_EDGES = 320000
HID = 128
D_OUT = 64
N_GRAPHS = 64

NC = 2
NS = 16
NW = NC * NS
EPW = N_EDGES // NW
CH = 80
NCHUNK = EPW // CH
RPT = 632
N_PAD = NS * RPT


NBUF = 3


def _sc_aggr_body(h_hbm, src_hbm, dst_hbm, zeros_hbm, out_hbm,
                  sidx_v, didx_v, rows_v, aggr_sh,
                  sem0, sem1, sem2, isem0, isem1, isem2,
                  jsem0, jsem1, jsem2):
    sems = (sem0, sem1, sem2)
    isems = (isem0, isem1, isem2)
    jsems = (jsem0, jsem1, jsem2)
    cid = lax.axis_index("c")
    sid = lax.axis_index("s")
    wid = cid * NS + sid

    pltpu.sync_copy(zeros_hbm, aggr_sh.at[pl.ds(sid * RPT, RPT)])
    plsc.subcore_barrier()

    for b in range(NBUF):
        pltpu.async_copy(dst_hbm.at[wid, b], didx_v.at[b], isems[b])
        pltpu.async_copy(src_hbm.at[wid, b], sidx_v.at[b], jsems[b])
    for b in range(NBUF - 1):
        pltpu.make_async_copy(src_hbm.at[wid, b], sidx_v.at[b],
                              jsems[b]).wait()
        pltpu.async_copy(h_hbm.at[sidx_v.at[b, 0]], rows_v.at[b], sems[b])

    def step(j, b):
        pltpu.make_async_copy(dst_hbm.at[wid, j], didx_v.at[b],
                              isems[b]).wait()
        pltpu.make_async_copy(h_hbm.at[sidx_v.at[b, 0]], rows_v.at[b],
                              sems[b]).wait()
        pltpu.sync_copy(rows_v.at[b], aggr_sh.at[didx_v.at[b, 0]], add=True)
        nj = j + NBUF

        @pl.when(nj < NCHUNK)
        def _():
            pltpu.async_copy(dst_hbm.at[wid, nj], didx_v.at[b], isems[b])
            pltpu.async_copy(src_hbm.at[wid, nj], sidx_v.at[b], jsems[b])
        ng = j + NBUF - 1
        b2 = (b + NBUF - 1) % NBUF

        @pl.when(ng < NCHUNK)
        def _():
            pltpu.make_async_copy(src_hbm.at[wid, ng], sidx_v.at[b2],
                                  jsems[b2]).wait()
            pltpu.async_copy(h_hbm.at[sidx_v.at[b2, 0]], rows_v.at[b2],
                             sems[b2])

    def outer(o, carry):
        for b in range(NBUF):
            step(o * NBUF + b, b)
        return carry

    lax.fori_loop(0, NCHUNK // NBUF, outer, 0, unroll=False)
    for j in range((NCHUNK // NBUF) * NBUF, NCHUNK):
        step(j, j % NBUF)
    plsc.subcore_barrier()
    pltpu.sync_copy(aggr_sh.at[pl.ds(sid * RPT, RPT)],
                    out_hbm.at[pl.ds(cid * N_PAD + sid * RPT, RPT)])


_sc_aggr_cache = []


def _sc_aggr(*args):
    if not _sc_aggr_cache:
        _sc_aggr_cache.append(functools.partial(
            pl.kernel,
            out_type=jax.ShapeDtypeStruct((NC * N_PAD, HID), jnp.float32),
            mesh=plsc.VectorSubcoreMesh(core_axis_name="c",
                                        subcore_axis_name="s"),
            scratch_types=[
                pltpu.VMEM((NBUF, 1, CH), jnp.int32),
                pltpu.VMEM((NBUF, 1, CH), jnp.int32),
                pltpu.VMEM((NBUF, CH, HID), jnp.float32),
                pltpu.VMEM_SHARED((N_PAD, HID), jnp.float32),
            ] + [pltpu.SemaphoreType.DMA] * (3 * NBUF),
        )(_sc_aggr_body))
    return _sc_aggr_cache[0](*args)


_dot = functools.partial(jnp.dot, preferred_element_type=jnp.float32)
_segdot = functools.partial(
    lax.dot_general, dimension_numbers=(((0,), (0,)), ((), ())),
    preferred_element_type=jnp.float32)


def _prep_body(b_ref, oh_ref, oht_ref, invc_ref):
    gid = lax.broadcasted_iota(jnp.int32, (N_GRAPHS, N_NODES), 0)
    oht = (gid == b_ref[...]).astype(jnp.float32)
    oht_ref[...] = oht
    oh_ref[...] = oht.T
    cnt = jnp.sum(oht, axis=1, keepdims=True)
    invc_ref[...] = 1.0 / jnp.maximum(cnt, 1.0)


_prep = pl.pallas_call(
    _prep_body,
    out_shape=(
        jax.ShapeDtypeStruct((N_NODES, N_GRAPHS), jnp.float32),
        jax.ShapeDtypeStruct((N_GRAPHS, N_NODES), jnp.float32),
        jax.ShapeDtypeStruct((N_GRAPHS, 1), jnp.float32),
    ),
)


def _gnorm_body(h_ref, pp_ref, w1_ref, b1_ref, w2_ref, b2_ref, eps_ref,
                oh_ref, oht_ref, invc_ref,
                gnw_ref, gnb_ref, gns_ref,
                pw1_ref, pb1_ref, pw2_ref, pb2_ref, hout_ref, zg_ref):
    out = (1.0 + eps_ref[0, 0]) * h_ref[...] \
        + pp_ref[0, :N_NODES] + pp_ref[1, :N_NODES]
    out = jnp.maximum(_dot(out, w1_ref[...]) + b1_ref[...], 0.0)
    out = _dot(out, w2_ref[...]) + b2_ref[...]
    onehot = oh_ref[...]
    oht = oht_ref[...]
    inv_cnt = invc_ref[...]

    mean = _dot(oht, out) * inv_cnt
    sub = out - _dot(onehot, mean) * gns_ref[...]
    var = _dot(oht, sub * sub) * inv_cnt
    inv_std = 1.0 / jnp.sqrt(var + 1e-08)
    res = gnw_ref[...] * sub * _dot(onehot, inv_std) + gnb_ref[...]
    hn = jnp.maximum(res, 0.0)
    hout_ref[...] = hn
    z = jnp.maximum(_dot(hn, pw1_ref[...]) + pb1_ref[...], 0.0)
    z = _dot(z, pw2_ref[...]) + pb2_ref[...]
    zg_ref[...] = _dot(oht, z)


_gnorm = pl.pallas_call(
    _gnorm_body,
    out_shape=(
        jax.ShapeDtypeStruct((N_NODES, HID), jnp.float32),
        jax.ShapeDtypeStruct((N_GRAPHS, D_OUT), jnp.float32),
    ),
)


def kernel(x, edge_index, batch, params):
    src = edge_index[0].reshape(NW, NCHUNK, 1, CH)
    dst = edge_index[1].reshape(NW, NCHUNK, 1, CH)
    zeros_blk = jnp.zeros((RPT, HID), jnp.float32)
    oh, oht, invc = _prep(batch.reshape(1, N_NODES))

    h = x
    z_cat = []
    for l in range(3):
        p = params[f'l{l}']
        partials = _sc_aggr(h, src, dst, zeros_blk).reshape(NC, N_PAD, HID)
        h, zg = _gnorm(
            h, partials,
            p['w1'], p['b1'].reshape(1, HID),
            p['w2'], p['b2'].reshape(1, HID),
            p['eps'].reshape(1, 1),
            oh, oht, invc,
            p['gn_weight'].reshape(1, HID),
            p['gn_bias'].reshape(1, HID),
            p['gn_scale'].reshape(1, HID),
            p['pw1'], p['pb1'].reshape(1, HID),
            p['pw2'], p['pb2'].reshape(1, D_OUT),
        )
        z_cat.append(zg)
    return jnp.concatenate(z_cat, axis=-1)
